# baseline (speedup 1.0000x reference)
import functools

import jax
import jax.numpy as jnp
from jax import lax
from jax.experimental import pallas as pl
from jax.experimental.pallas import tpu as pltpu
from jax.experimental.pallas import tpu_sc as plsc

_B = 16384
_D = 64
_NC = 2
_NS = 16
_NW = _NC * _NS
_RPW = _B // _NW
_CHUNK = 128
_NCHUNK = _RPW // _CHUNK


def _body(uidx_hbm, vidx_hbm, utab_hbm, vtab_hbm, params_hbm, out_hbm,
          idx_u, idx_v, rows_u, rows_v, params_v, out_v, sem):
    wid = lax.axis_index("s") * _NC + lax.axis_index("c")
    base = wid * _RPW

    pltpu.sync_copy(params_hbm, params_v)
    pltpu.sync_copy(uidx_hbm.at[pl.ds(base, _RPW)], idx_u)
    pltpu.sync_copy(vidx_hbm.at[pl.ds(base, _RPW)], idx_v)

    copies = []
    for c in range(_NCHUNK):
        sl = pl.ds(c * _CHUNK, _CHUNK)
        copies.append(pltpu.async_copy(
            utab_hbm.at[idx_u.at[sl]], rows_u.at[sl, :], sem))
        copies.append(pltpu.async_copy(
            vtab_hbm.at[idx_v.at[sl]], rows_v.at[sl, :], sem))
    for cp in copies:
        cp.wait()

    bias_u = params_v[pl.ds(2 * 16 * _D, 16)]
    bias_v = params_v[pl.ds(2 * 16 * _D + 16, 16)]
    lanes = lax.iota(jnp.int32, 16)

    def blk_body(blk, _):
        rid = blk * 16 + lanes

        def d_body(d, accs):
            au, av = accs
            col = jnp.broadcast_to(d, (16,))
            gu = plsc.load_gather(rows_u, [rid, col])
            gv = plsc.load_gather(rows_v, [rid, col])
            au = au + gu * params_v[pl.ds(d * 16, 16)]
            av = av + gv * params_v[pl.ds((_D + d) * 16, 16)]
            return (au, av)

        zero = jnp.zeros((16,), jnp.float32)
        au, av = lax.fori_loop(0, _D, d_body, (zero, zero))
        out_v[pl.ds(blk * 16, 16)] = (au + bias_u) * (av + bias_v)
        return 0

    lax.fori_loop(0, _RPW // 16, blk_body, 0)
    pltpu.sync_copy(out_v, out_hbm.at[pl.ds(base, _RPW)])


@jax.jit
def _gmf(uidx, vidx, utab, vtab, params):
    mesh = plsc.VectorSubcoreMesh(
        core_axis_name="c", subcore_axis_name="s",
        num_cores=_NC, num_subcores=_NS)
    f = pl.kernel(
        _body,
        out_type=jax.ShapeDtypeStruct((_B,), jnp.float32),
        mesh=mesh,
        scratch_types=[
            pltpu.VMEM((_RPW,), jnp.int32),
            pltpu.VMEM((_RPW,), jnp.int32),
            pltpu.VMEM((_RPW, _D), jnp.float32),
            pltpu.VMEM((_RPW, _D), jnp.float32),
            pltpu.VMEM((2 * 16 * _D + 32,), jnp.float32),
            pltpu.VMEM((_RPW,), jnp.float32),
            pltpu.SemaphoreType.DMA,
        ],
        compiler_params=pltpu.CompilerParams(
            needs_layout_passes=False, use_tc_tiling_on_sc=False),
    )
    return f(uidx, vidx, utab, vtab, params)


def kernel(user_indices, item_indices, user_table, item_table,
           fc_user_w, fc_user_b, fc_item_w, fc_item_b):
    wtab_u = jnp.broadcast_to(fc_user_w.reshape(_D, 1), (_D, 16)).reshape(-1)
    wtab_v = jnp.broadcast_to(fc_item_w.reshape(_D, 1), (_D, 16)).reshape(-1)
    params = jnp.concatenate([
        wtab_u.astype(jnp.float32),
        wtab_v.astype(jnp.float32),
        jnp.broadcast_to(fc_user_b.reshape(1).astype(jnp.float32), (16,)),
        jnp.broadcast_to(fc_item_b.reshape(1).astype(jnp.float32), (16,)),
    ])
    out = _gmf(user_indices.astype(jnp.int32), item_indices.astype(jnp.int32),
               user_table, item_table, params)
    return out.reshape(_B, 1)
